# baseline (speedup 1.0000x reference)
import functools

import jax
import jax.numpy as jnp
from jax import lax
from jax.experimental import pallas as pl
from jax.experimental.pallas import tpu as pltpu
from jax.experimental.pallas import tpu_sc as plsc


def _encode_vq_body(x_ref, u_ref, cbt_ref, csq_ref, z_ref, idx_ref):
    z = jnp.dot(x_ref[...], u_ref[...], preferred_element_type=jnp.float32)
    z_ref[...] = z
    m = jnp.dot(z, cbt_ref[...], preferred_element_type=jnp.float32)
    z_sq = jnp.sum(z * z, axis=1, keepdims=True)
    d2 = (z_sq - 2.0 * m) + csq_ref[...]
    idx_ref[...] = jnp.argmin(d2, axis=1).astype(jnp.int32)[:, None]


def _csq_body(cbt_ref, csq_ref):
    c = cbt_ref[...]
    csq_ref[...] = jnp.sum(c * c, axis=0, keepdims=True)


def _make_sc_gather(cb_rows, k_lat, batch):
    info = plsc.get_sparse_core_info()
    nw = info.num_cores * info.num_subcores
    rows_per_w = batch // nw
    chunk = 128
    nch = rows_per_w // chunk
    mesh = plsc.VectorSubcoreMesh(core_axis_name="c", subcore_axis_name="s")

    @functools.partial(
        pl.kernel,
        mesh=mesh,
        out_type=jax.ShapeDtypeStruct((batch, k_lat), jnp.float32),
        scratch_types=[
            pltpu.VMEM((nch, chunk), jnp.int32),
            pltpu.VMEM((2, chunk, k_lat), jnp.float32),
            pltpu.SemaphoreType.DMA,
            pltpu.SemaphoreType.DMA,
        ],
    )
    def gather(cb_hbm, idx_hbm, zq_hbm, idx_v, rows_v, sem0, sem1):
        wid = lax.axis_index("s") * info.num_cores + lax.axis_index("c")
        base = wid * rows_per_w
        pltpu.sync_copy(idx_hbm.at[wid], idx_v)
        sems = (sem0, sem1)
        pending = {}
        for j in range(min(2, nch)):
            pending[j] = pltpu.async_copy(
                cb_hbm.at[idx_v.at[j]], rows_v.at[j % 2], sems[j % 2]
            )
        for j in range(nch):
            pending[j].wait()
            pltpu.sync_copy(rows_v.at[j % 2], zq_hbm.at[pl.ds(base + j * chunk, chunk)])
            if j + 2 < nch:
                pending[j + 2] = pltpu.async_copy(
                    cb_hbm.at[idx_v.at[j + 2]], rows_v.at[j % 2], sems[j % 2]
                )

    return gather


def _decode_body(z_ref, zq_ref, ukt_ref, zst_ref, xr_ref):
    z = z_ref[...]
    zst = z + (zq_ref[:, : z.shape[1]] - z)
    zst_ref[...] = zst
    xr_ref[...] = jnp.dot(zst, ukt_ref[...], preferred_element_type=jnp.float32)


def kernel(x, U_k, codebook):
    b, d = x.shape
    k_lat = U_k.shape[1]
    cb = codebook.shape[0]

    cbt = codebook.T

    csq = pl.pallas_call(
        _csq_body,
        out_shape=jax.ShapeDtypeStruct((1, cb), jnp.float32),
    )(cbt)

    bm = 512
    z, idx2d = pl.pallas_call(
        _encode_vq_body,
        grid=(b // bm,),
        in_specs=[
            pl.BlockSpec((bm, d), lambda i: (i, 0)),
            pl.BlockSpec((d, k_lat), lambda i: (0, 0)),
            pl.BlockSpec((k_lat, cb), lambda i: (0, 0)),
            pl.BlockSpec((1, cb), lambda i: (0, 0)),
        ],
        out_specs=[
            pl.BlockSpec((bm, k_lat), lambda i: (i, 0)),
            pl.BlockSpec((bm, 1), lambda i: (i, 0)),
        ],
        out_shape=[
            jax.ShapeDtypeStruct((b, k_lat), jnp.float32),
            jax.ShapeDtypeStruct((b, 1), jnp.int32),
        ],
        compiler_params=pltpu.CompilerParams(
            dimension_semantics=("arbitrary",),
        ),
    )(x, U_k, cbt, csq)

    indices = idx2d.reshape(b)

    k_pad = 128
    cb_padded = jnp.pad(codebook, ((0, 0), (0, k_pad - k_lat)))
    nw = 32
    z_q = jnp.take(cb_padded, indices, axis=0)

    bm3 = 512
    z_q_st, x_recon = pl.pallas_call(
        _decode_body,
        grid=(b // bm3,),
        in_specs=[
            pl.BlockSpec((bm3, k_lat), lambda i: (i, 0)),
            pl.BlockSpec((bm3, k_pad), lambda i: (i, 0)),
            pl.BlockSpec((k_lat, d), lambda i: (0, 0)),
        ],
        out_specs=[
            pl.BlockSpec((bm3, k_lat), lambda i: (i, 0)),
            pl.BlockSpec((bm3, d), lambda i: (i, 0)),
        ],
        out_shape=[
            jax.ShapeDtypeStruct((b, k_lat), jnp.float32),
            jax.ShapeDtypeStruct((b, d), jnp.float32),
        ],
        compiler_params=pltpu.CompilerParams(
            dimension_semantics=("arbitrary",),
        ),
    )(z, z_q, U_k.T)

    return (x_recon, z, z_q_st, indices)


# speedup vs baseline: 1.2071x; 1.2071x over previous
import functools

import jax
import jax.numpy as jnp
from jax import lax
from jax.experimental import pallas as pl
from jax.experimental.pallas import tpu as pltpu
from jax.experimental.pallas import tpu_sc as plsc


def _encode_vq_body(x_ref, u_ref, cbt_ref, csq_ref, z_ref, idx_ref):
    z = jnp.dot(x_ref[...], u_ref[...], preferred_element_type=jnp.float32)
    z_ref[...] = z
    m = jnp.dot(z, cbt_ref[...], preferred_element_type=jnp.float32)
    z_sq = jnp.sum(z * z, axis=1, keepdims=True)
    d2 = (z_sq - 2.0 * m) + csq_ref[...]
    idx_ref[...] = jnp.argmin(d2, axis=1).astype(jnp.int32)[:, None]


def _csq_body(cbt_ref, csq_ref):
    c = cbt_ref[...]
    csq_ref[...] = jnp.sum(c * c, axis=0, keepdims=True)


def _make_sc_gather(cb_rows, k_lat, batch):
    info = plsc.get_sparse_core_info()
    nw = info.num_cores * info.num_subcores
    rows_per_w = batch // nw
    chunk = 128
    nch = rows_per_w // chunk
    mesh = plsc.VectorSubcoreMesh(core_axis_name="c", subcore_axis_name="s")

    @functools.partial(
        pl.kernel,
        mesh=mesh,
        out_type=jax.ShapeDtypeStruct((batch, k_lat), jnp.float32),
        scratch_types=[
            pltpu.VMEM((nch, chunk), jnp.int32),
            pltpu.VMEM((2, chunk, k_lat), jnp.float32),
            pltpu.SemaphoreType.DMA,
            pltpu.SemaphoreType.DMA,
        ],
    )
    def gather(cb_hbm, idx_hbm, zq_hbm, idx_v, rows_v, sem0, sem1):
        wid = lax.axis_index("s") * info.num_cores + lax.axis_index("c")
        base = wid * rows_per_w
        pltpu.sync_copy(idx_hbm.at[wid], idx_v)
        sems = (sem0, sem1)
        pending = {}
        for j in range(min(2, nch)):
            pending[j] = pltpu.async_copy(
                cb_hbm.at[idx_v.at[j]], rows_v.at[j % 2], sems[j % 2]
            )
        for j in range(nch):
            pending[j].wait()
            pltpu.sync_copy(rows_v.at[j % 2], zq_hbm.at[pl.ds(base + j * chunk, chunk)])
            if j + 2 < nch:
                pending[j + 2] = pltpu.async_copy(
                    cb_hbm.at[idx_v.at[j + 2]], rows_v.at[j % 2], sems[j % 2]
                )

    return gather


def _decode_body(z_ref, zq_ref, ukt_ref, zst_ref, xr_ref):
    z = z_ref[...]
    zst = z + (zq_ref[:, : z.shape[1]] - z)
    zst_ref[...] = zst
    xr_ref[...] = jnp.dot(zst, ukt_ref[...], preferred_element_type=jnp.float32)


def kernel(x, U_k, codebook):
    b, d = x.shape
    k_lat = U_k.shape[1]
    cb = codebook.shape[0]

    cbt = codebook.T

    csq = pl.pallas_call(
        _csq_body,
        out_shape=jax.ShapeDtypeStruct((1, cb), jnp.float32),
    )(cbt)

    bm = 512
    z, idx2d = pl.pallas_call(
        _encode_vq_body,
        grid=(b // bm,),
        in_specs=[
            pl.BlockSpec((bm, d), lambda i: (i, 0)),
            pl.BlockSpec((d, k_lat), lambda i: (0, 0)),
            pl.BlockSpec((k_lat, cb), lambda i: (0, 0)),
            pl.BlockSpec((1, cb), lambda i: (0, 0)),
        ],
        out_specs=[
            pl.BlockSpec((bm, k_lat), lambda i: (i, 0)),
            pl.BlockSpec((bm, 1), lambda i: (i, 0)),
        ],
        out_shape=[
            jax.ShapeDtypeStruct((b, k_lat), jnp.float32),
            jax.ShapeDtypeStruct((b, 1), jnp.int32),
        ],
        compiler_params=pltpu.CompilerParams(
            dimension_semantics=("arbitrary",),
        ),
    )(x, U_k, cbt, csq)

    indices = idx2d.reshape(b)

    k_pad = 128
    cb_padded = jnp.pad(codebook, ((0, 0), (0, k_pad - k_lat)))
    nw = 32
    z_q = _make_sc_gather(cb, k_pad, b)(cb_padded, idx2d.reshape(nw, -1, 128))

    bm3 = 512
    z_q_st, x_recon = pl.pallas_call(
        _decode_body,
        grid=(b // bm3,),
        in_specs=[
            pl.BlockSpec((bm3, k_lat), lambda i: (i, 0)),
            pl.BlockSpec((bm3, k_pad), lambda i: (i, 0)),
            pl.BlockSpec((k_lat, d), lambda i: (0, 0)),
        ],
        out_specs=[
            pl.BlockSpec((bm3, k_lat), lambda i: (i, 0)),
            pl.BlockSpec((bm3, d), lambda i: (i, 0)),
        ],
        out_shape=[
            jax.ShapeDtypeStruct((b, k_lat), jnp.float32),
            jax.ShapeDtypeStruct((b, d), jnp.float32),
        ],
        compiler_params=pltpu.CompilerParams(
            dimension_semantics=("arbitrary",),
        ),
    )(z, z_q, U_k.T)

    return (x_recon, z, z_q_st, indices)
